# baseline (speedup 1.0000x reference)
import jax
import jax.numpy as jnp
from jax import lax
from jax.experimental import pallas as pl
from jax.experimental.pallas import tpu as pltpu
from jax.experimental.pallas import tpu_sc as plsc

H = W = 384
N_IMG = 4
C_CH = 96
M_PATCH = 64
PS = 16
SY = 18
SX = 32
PW = 18
NWORKERS = 32
TASKS_PER_W = (N_IMG * M_PATCH) // NWORKERS


def _axis_setup(off_splat, center_scalar_f32):
    ii = lax.convert_element_type(lax.iota(jnp.int32, 16), jnp.float32)
    coord = center_scalar_f32 - 8.0 + ii
    cn = 2.0 * (coord / 383.0) - 1.0
    g = cn + off_splat
    ic = ((g + 1.0) * 384.0 - 1.0) * 0.5
    ic = jnp.clip(ic, -1e4, 1e4)
    t0 = lax.convert_element_type(ic, jnp.int32)
    t0 = jnp.where(lax.convert_element_type(t0, jnp.float32) > ic, t0 - 1, t0)
    f0 = lax.convert_element_type(t0, jnp.float32)
    w1 = ic - f0
    w0 = 1.0 - w1
    a0 = w0 * jnp.where((t0 >= 0) & (t0 <= W - 1), 1.0, 0.0)
    a1 = w1 * jnp.where((t0 >= -1) & (t0 <= W - 2), 1.0, 0.0)
    c0 = jnp.clip(t0, 0, W - 1)
    c1 = jnp.clip(t0 + 1, 0, W - 1)
    return t0, c0, c1, a0, a1


def _sc_body(x_hbm, off_hbm, out_hbm, reg, prow, outb, offv, xit, axt):
    cid = lax.axis_index("c")
    sid = lax.axis_index("s")
    wid = sid * 2 + cid

    pltpu.sync_copy(off_hbm, offv)
    iota16 = lax.iota(jnp.int32, 16)
    zv16 = jnp.zeros((16,), jnp.int32)

    def task_body(k, _):
        t = wid * TASKS_PER_W + k
        n = lax.div(t, M_PATCH)
        m = lax.rem(t, M_PATCH)
        mh = lax.div(m, 8)
        mw = lax.rem(m, 8)

        def center(mk):
            mkf = lax.convert_element_type(mk + 2, jnp.float32)
            return mkf * jnp.float32(384.0 / 11.0)

        mv = jnp.full((16,), m, jnp.int32)
        off0 = plsc.load_gather(offv, [mv, zv16])
        off1 = plsc.load_gather(offv, [mv, zv16 + 1])

        x0i, x0c, x1c, ax0, ax1 = _axis_setup(off0, center(mh))
        xmin = jnp.min(x0i)
        xb = pl.multiple_of(
            jnp.clip(lax.bitwise_and(xmin, jnp.int32(-8)), 0, W - SX), 8
        )
        rel0 = jnp.clip(xmin - xb, 0, SX - PW)
        x0r = jnp.clip(x0c - xb, rel0, rel0 + PW - 1)
        x1r = jnp.clip(x1c - xb, rel0, rel0 + PW - 1)
        y0i, y0c, y1c, ay0, ay1 = _axis_setup(off1, center(mw))
        ymin = jnp.min(y0i)
        yb = jnp.clip(ymin, 0, H - SY)
        y0r = jnp.clip(y0c - yb, 0, SY - 1)
        y1r = jnp.clip(y1c - yb, 0, SY - 1)

        xit[0, :] = (x0r - rel0) * 16
        xit[1, :] = (x1r - rel0) * 16
        axt[0, :] = ax0
        axt[1, :] = ax1

        @plsc.parallel_loop(0, 16, unroll=4)
        def table_body(i):
            fi = jnp.full((16,), i, jnp.int32)
            xit[2 + i, :] = plsc.load_gather(xit.at[0], [fi]) + iota16
            xit[18 + i, :] = plsc.load_gather(xit.at[1], [fi]) + iota16
            axt[2 + i, :] = plsc.load_gather(axt.at[0], [fi])
            axt[18 + i, :] = plsc.load_gather(axt.at[1], [fi])

        pltpu.sync_copy(
            x_hbm.at[pl.ds(n * C_CH, C_CH), pl.ds(yb, SY), pl.ds(xb, SX)], reg
        )

        @plsc.parallel_loop(0, C_CH * PW, unroll=12)
        def p_body(q):
            c = lax.div(q, PW)
            dx = lax.rem(q, PW)
            cv = jnp.full((16,), c, jnp.int32)
            colv = jnp.full((16,), rel0 + dx, jnp.int32)
            g0 = plsc.load_gather(reg, [cv, y0r, colv])
            g1 = plsc.load_gather(reg, [cv, y1r, colv])
            prow[pl.ds(q * 16, 16)] = ay0 * g0 + ay1 * g1

        def i_body(i, _):
            t0 = xit[2 + i, :]
            t1 = xit[18 + i, :]
            w0 = axt[2 + i, :]
            w1 = axt[18 + i, :]
            ib = i * 16

            @plsc.parallel_loop(0, C_CH, unroll=12)
            def o_body(c):
                pb = jnp.full((16,), c * (PW * 16), jnp.int32)
                p0 = plsc.load_gather(prow, [t0 + pb])
                p1 = plsc.load_gather(prow, [t1 + pb])
                outb[c, pl.ds(ib, 16)] = w0 * p0 + w1 * p1

            return 0

        lax.fori_loop(0, 16, i_body, 0, unroll=1)

        pltpu.sync_copy(outb, out_hbm.at[n, m])
        return 0

    lax.fori_loop(0, TASKS_PER_W, task_body, 0, unroll=1)


def kernel(x, offset):
    x3 = x.reshape(N_IMG * C_CH, H, W)

    mesh = plsc.VectorSubcoreMesh(
        core_axis_name="c", subcore_axis_name="s", num_cores=2, num_subcores=16
    )
    run = pl.kernel(
        _sc_body,
        out_type=jax.ShapeDtypeStruct((N_IMG, M_PATCH, C_CH, PS * PS), jnp.float32),
        mesh=mesh,
        compiler_params=pltpu.CompilerParams(
            use_tc_tiling_on_sc=False,
            needs_layout_passes=False,
            disable_bounds_checks=True,
            disable_semaphore_checks=True,
            skip_device_barrier=True,
        ),
        scratch_types=[
            pltpu.VMEM((C_CH, SY, SX), jnp.float32),
            pltpu.VMEM((C_CH * PW * 16,), jnp.float32),
            pltpu.VMEM((C_CH, PS * PS), jnp.float32),
            pltpu.VMEM((M_PATCH, 2), jnp.float32),
            pltpu.VMEM((34, 16), jnp.int32),
            pltpu.VMEM((34, 16), jnp.float32),
        ],
    )
    out = run(x3, offset)
    return out.reshape(N_IMG, M_PATCH, C_CH, PS, PS)


# speedup vs baseline: 1.0734x; 1.0734x over previous
import jax
import jax.numpy as jnp
from jax import lax
from jax.experimental import pallas as pl
from jax.experimental.pallas import tpu as pltpu
from jax.experimental.pallas import tpu_sc as plsc

H = W = 384
N_IMG = 4
C_CH = 96
M_PATCH = 64
PS = 16
SY = 18
SX = 32
PW = 18
NWORKERS = 32
TASKS_PER_W = (N_IMG * M_PATCH) // NWORKERS


def _axis_setup(off_splat, center_scalar_f32):
    ii = lax.convert_element_type(lax.iota(jnp.int32, 16), jnp.float32)
    coord = center_scalar_f32 - 8.0 + ii
    cn = 2.0 * (coord / 383.0) - 1.0
    g = cn + off_splat
    ic = ((g + 1.0) * 384.0 - 1.0) * 0.5
    ic = jnp.clip(ic, -1e4, 1e4)
    t0 = lax.convert_element_type(ic, jnp.int32)
    t0 = jnp.where(lax.convert_element_type(t0, jnp.float32) > ic, t0 - 1, t0)
    f0 = lax.convert_element_type(t0, jnp.float32)
    w1 = ic - f0
    w0 = 1.0 - w1
    a0 = w0 * jnp.where((t0 >= 0) & (t0 <= W - 1), 1.0, 0.0)
    a1 = w1 * jnp.where((t0 >= -1) & (t0 <= W - 2), 1.0, 0.0)
    c0 = jnp.clip(t0, 0, W - 1)
    c1 = jnp.clip(t0 + 1, 0, W - 1)
    return t0, c0, c1, a0, a1


def _sc_body(x_hbm, off_hbm, out_hbm, reg, prow, outb, offv, xit, axt):
    cid = lax.axis_index("c")
    sid = lax.axis_index("s")
    wid = sid * 2 + cid

    pltpu.sync_copy(off_hbm, offv)
    iota16 = lax.iota(jnp.int32, 16)
    zv16 = jnp.zeros((16,), jnp.int32)

    def task_body(k, _):
        t = wid * TASKS_PER_W + k
        n = lax.div(t, M_PATCH)
        m = lax.rem(t, M_PATCH)
        mh = lax.div(m, 8)
        mw = lax.rem(m, 8)

        def center(mk):
            mkf = lax.convert_element_type(mk + 2, jnp.float32)
            return mkf * jnp.float32(384.0 / 11.0)

        mv = jnp.full((16,), m, jnp.int32)
        off0 = plsc.load_gather(offv, [mv, zv16])
        off1 = plsc.load_gather(offv, [mv, zv16 + 1])

        x0i, x0c, x1c, ax0, ax1 = _axis_setup(off0, center(mh))
        xmin = jnp.min(x0i)
        xb = pl.multiple_of(
            jnp.clip(lax.bitwise_and(xmin, jnp.int32(-8)), 0, W - SX), 8
        )
        rel0 = jnp.clip(xmin - xb, 0, SX - PW)
        x0r = jnp.clip(x0c - xb, rel0, rel0 + PW - 1)
        x1r = jnp.clip(x1c - xb, rel0, rel0 + PW - 1)
        y0i, y0c, y1c, ay0, ay1 = _axis_setup(off1, center(mw))
        ymin = jnp.min(y0i)
        yb = jnp.clip(ymin, 0, H - SY)
        y0r = jnp.clip(y0c - yb, 0, SY - 1)
        y1r = jnp.clip(y1c - yb, 0, SY - 1)

        xit[0, :] = (x0r - rel0) * 16
        xit[1, :] = (x1r - rel0) * 16
        axt[0, :] = ax0
        axt[1, :] = ax1

        @plsc.parallel_loop(0, 16, unroll=4)
        def table_body(i):
            fi = jnp.full((16,), i, jnp.int32)
            xit[2 + i, :] = plsc.load_gather(xit.at[0], [fi]) + iota16
            xit[18 + i, :] = plsc.load_gather(xit.at[1], [fi]) + iota16
            axt[2 + i, :] = plsc.load_gather(axt.at[0], [fi])
            axt[18 + i, :] = plsc.load_gather(axt.at[1], [fi])

        pltpu.sync_copy(
            x_hbm.at[pl.ds(n * C_CH, C_CH), pl.ds(yb, SY), pl.ds(xb, SX)], reg
        )

        @plsc.parallel_loop(0, C_CH * PW, unroll=6)
        def p_body(q):
            c = lax.div(q, PW)
            dx = lax.rem(q, PW)
            cv = jnp.full((16,), c, jnp.int32)
            colv = jnp.full((16,), rel0 + dx, jnp.int32)
            g0 = plsc.load_gather(reg, [cv, y0r, colv])
            g1 = plsc.load_gather(reg, [cv, y1r, colv])
            prow[pl.ds(q * 16, 16)] = ay0 * g0 + ay1 * g1

        def i_body(i, _):
            t0 = xit[2 + i, :]
            t1 = xit[18 + i, :]
            w0 = axt[2 + i, :]
            w1 = axt[18 + i, :]
            ib = i * 16

            @plsc.parallel_loop(0, C_CH, unroll=8)
            def o_body(c):
                pb = jnp.full((16,), c * (PW * 16), jnp.int32)
                p0 = plsc.load_gather(prow, [t0 + pb])
                p1 = plsc.load_gather(prow, [t1 + pb])
                outb[c, pl.ds(ib, 16)] = w0 * p0 + w1 * p1

            return 0

        lax.fori_loop(0, 16, i_body, 0, unroll=1)

        pltpu.sync_copy(outb, out_hbm.at[n, m])
        return 0

    lax.fori_loop(0, TASKS_PER_W, task_body, 0, unroll=1)


def kernel(x, offset):
    x3 = x.reshape(N_IMG * C_CH, H, W)

    mesh = plsc.VectorSubcoreMesh(
        core_axis_name="c", subcore_axis_name="s", num_cores=2, num_subcores=16
    )
    run = pl.kernel(
        _sc_body,
        out_type=jax.ShapeDtypeStruct((N_IMG, M_PATCH, C_CH, PS * PS), jnp.float32),
        mesh=mesh,
        compiler_params=pltpu.CompilerParams(
            use_tc_tiling_on_sc=False,
            needs_layout_passes=False,
            disable_bounds_checks=True,
            disable_semaphore_checks=True,
            skip_device_barrier=True,
        ),
        scratch_types=[
            pltpu.VMEM((C_CH, SY, SX), jnp.float32),
            pltpu.VMEM((C_CH * PW * 16,), jnp.float32),
            pltpu.VMEM((C_CH, PS * PS), jnp.float32),
            pltpu.VMEM((M_PATCH, 2), jnp.float32),
            pltpu.VMEM((34, 16), jnp.int32),
            pltpu.VMEM((34, 16), jnp.float32),
        ],
    )
    out = run(x3, offset)
    return out.reshape(N_IMG, M_PATCH, C_CH, PS, PS)
